# baseline (speedup 1.0000x reference)
import jax
import jax.numpy as jnp
from jax.experimental import pallas as pl

_N = 1024
_K = 16
_RB = 256
_GO = 32
_HP = jax.lax.Precision.HIGHEST
_DP = jax.lax.Precision.DEFAULT


def _knn_adj_kernel(hall_ref, hrow_ref, m_ref):
    h = hall_ref[0]
    hb = hrow_ref[0]
    sq_all = jnp.sum(h * h, axis=1)[None, :]
    sq_b = jnp.sum(hb * hb, axis=1)[:, None]
    cross = jax.lax.dot_general(hb, h, (((1,), (1,)), ((), ())),
                                precision=_DP)
    d2 = (sq_b + sq_all) - 2.0 * cross
    r0 = pl.program_id(1) * _RB
    rows = r0 + jax.lax.broadcasted_iota(jnp.int32, (_RB, _N), 0)
    cols = jax.lax.broadcasted_iota(jnp.int32, (_RB, _N), 1)
    diag = rows == cols
    d2 = jnp.where(diag, d2 + 1e9, d2)
    colf = cols.astype(jnp.float32)
    acc = jnp.zeros((_RB, _N), jnp.float32)
    for _ in range(_K):
        mn = jnp.min(d2, axis=1, keepdims=True)
        eq = d2 == mn
        amin = jnp.min(jnp.where(eq, colf, float(_N)), axis=1, keepdims=True)
        hit = colf == amin
        acc = acc + jnp.where(hit, 1.0 / _K, 0.0)
        d2 = jnp.where(hit, jnp.inf, d2)
    m_ref[0] = acc.astype(jnp.bfloat16)


def _conv_kernel(h0_ref, m_ref, w1_ref, b1_ref, w2_ref, b2_ref, w3_ref,
                 b3_ref, out_ref):
    h = h0_ref[0]
    m = m_ref[0]
    dn = (((1,), (0,)), ((), ()))
    for w_ref, b_ref in ((w1_ref, b1_ref), (w2_ref, b2_ref), (w3_ref, b3_ref)):
        hh = h.astype(jnp.bfloat16)
        r1 = h - hh.astype(jnp.float32)
        hm = r1.astype(jnp.bfloat16)
        hl = (r1 - hm.astype(jnp.float32)).astype(jnp.bfloat16)
        agg = (jax.lax.dot_general(m, hh, dn, precision=_DP,
                                   preferred_element_type=jnp.float32)
               + jax.lax.dot_general(m, hm, dn, precision=_DP,
                                     preferred_element_type=jnp.float32)
               + jax.lax.dot_general(m, hl, dn, precision=_DP,
                                     preferred_element_type=jnp.float32))
        hp = h + agg
        h = jnp.tanh(jax.lax.dot_general(
            hp, w_ref[...], (((1,), (0,)), ((), ())),
            precision=_DP) + b_ref[...])
    out_ref[0] = h


def _pool_kernel(h_ref, out_ref):
    h = h_ref[0]
    ch = h.shape[1]
    e = (jax.lax.broadcasted_iota(jnp.int32, (1, ch), 1) == (ch - 1))
    e = e.astype(jnp.float32)
    scores = jax.lax.dot_general(e, h, (((1,), (1,)), ((), ())),
                                 precision=_HP)
    cols = jax.lax.broadcasted_iota(jnp.int32, (1, _N), 1).astype(jnp.float32)
    rows32 = jax.lax.broadcasted_iota(jnp.int32, (_GO, 1), 0).astype(jnp.float32)
    ranks = jnp.zeros((1, _N), jnp.float32)
    for r in range(_GO):
        mx = jnp.max(scores, axis=1, keepdims=True)
        eq = scores == mx
        amin = jnp.min(jnp.where(eq, cols, float(_N)), axis=1, keepdims=True)
        hit = cols == amin
        ranks = ranks + jnp.where(hit, float(r + 1), 0.0)
        scores = jnp.where(hit, -jnp.inf, scores)
    sel = (ranks == rows32 + 1.0).astype(jnp.float32)
    out_ref[0] = jax.lax.dot_general(sel, h, (((1,), (0,)), ((), ())),
                                     precision=_HP)


def _fc1_kernel(x_ref, w_ref, b_ref, out_ref):
    k = pl.program_id(0)

    @pl.when(k == 0)
    def _init():
        out_ref[...] = jnp.zeros_like(out_ref)

    out_ref[...] += jax.lax.dot_general(
        x_ref[...], w_ref[...], (((1,), (0,)), ((), ())), precision=_DP)

    @pl.when(k == pl.num_programs(0) - 1)
    def _fin():
        v = out_ref[...] + b_ref[...]
        out_ref[...] = jnp.where(v >= 0, v, 0.01 * v)


def _fc2_kernel(x_ref, w_ref, b_ref, out_ref):
    out_ref[...] = jax.lax.dot_general(
        x_ref[...], w_ref[...], (((1,), (0,)), ((), ())),
        precision=_DP) + b_ref[...]


def kernel(x, W1, b1, W2, b2, W3, b3, Wc1, bc1, Wc2, bc2):
    bsz = x.shape[0]
    h0 = x.reshape(bsz, _N, -1)
    c = h0.shape[2]
    ch = W1.shape[1]

    m = pl.pallas_call(
        _knn_adj_kernel,
        grid=(bsz, _N // _RB),
        in_specs=[pl.BlockSpec((1, _N, c), lambda b, r: (b, 0, 0)),
                  pl.BlockSpec((1, _RB, c), lambda b, r: (b, r, 0))],
        out_specs=pl.BlockSpec((1, _RB, _N), lambda b, r: (b, r, 0)),
        out_shape=jax.ShapeDtypeStruct((bsz, _N, _N), jnp.bfloat16),
    )(h0, h0)

    h3 = pl.pallas_call(
        _conv_kernel,
        grid=(bsz,),
        in_specs=[pl.BlockSpec((1, _N, c), lambda b: (b, 0, 0)),
                  pl.BlockSpec((1, _N, _N), lambda b: (b, 0, 0)),
                  pl.BlockSpec(W1.shape, lambda b: (0, 0)),
                  pl.BlockSpec((1, ch), lambda b: (0, 0)),
                  pl.BlockSpec(W2.shape, lambda b: (0, 0)),
                  pl.BlockSpec((1, ch), lambda b: (0, 0)),
                  pl.BlockSpec(W3.shape, lambda b: (0, 0)),
                  pl.BlockSpec((1, ch), lambda b: (0, 0))],
        out_specs=pl.BlockSpec((1, _N, ch), lambda b: (b, 0, 0)),
        out_shape=jax.ShapeDtypeStruct((bsz, _N, ch), jnp.float32),
    )(h0, m, W1, b1.reshape(1, ch), W2, b2.reshape(1, ch),
      W3, b3.reshape(1, ch))

    pooled = pl.pallas_call(
        _pool_kernel,
        grid=(bsz,),
        in_specs=[pl.BlockSpec((1, _N, ch), lambda b: (b, 0, 0))],
        out_specs=pl.BlockSpec((1, _GO, ch), lambda b: (b, 0, 0)),
        out_shape=jax.ShapeDtypeStruct((bsz, _GO, ch), jnp.float32),
    )(h3)

    flat = pooled.reshape(bsz, _GO * ch)
    fc_in, fc_mid = Wc1.shape
    kn = 8
    kblk = fc_in // kn

    fc1 = pl.pallas_call(
        _fc1_kernel,
        grid=(kn,),
        in_specs=[pl.BlockSpec((bsz, kblk), lambda k: (0, k)),
                  pl.BlockSpec((kblk, fc_mid), lambda k: (k, 0)),
                  pl.BlockSpec((1, fc_mid), lambda k: (0, 0))],
        out_specs=pl.BlockSpec((bsz, fc_mid), lambda k: (0, 0)),
        out_shape=jax.ShapeDtypeStruct((bsz, fc_mid), jnp.float32),
    )(flat, Wc1, bc1.reshape(1, fc_mid))

    nclass = Wc2.shape[1]
    out = pl.pallas_call(
        _fc2_kernel,
        in_specs=[pl.BlockSpec((bsz, fc_mid), lambda: (0, 0)),
                  pl.BlockSpec(Wc2.shape, lambda: (0, 0)),
                  pl.BlockSpec((1, nclass), lambda: (0, 0))],
        out_specs=pl.BlockSpec((bsz, nclass), lambda: (0, 0)),
        out_shape=jax.ShapeDtypeStruct((bsz, nclass), jnp.float32),
    )(fc1, Wc2, bc2.reshape(1, nclass))
    return out


# speedup vs baseline: 1.0057x; 1.0057x over previous
import jax
import jax.numpy as jnp
from jax.experimental import pallas as pl

_N = 1024
_K = 16
_RB = 256
_GO = 32
_HP = jax.lax.Precision.HIGHEST
_DP = jax.lax.Precision.DEFAULT


def _knn_adj_kernel(hall_ref, hrow_ref, m_ref):
    h = hall_ref[0]
    hb = hrow_ref[0]
    sq_all = jnp.sum(h * h, axis=1)[None, :]
    sq_b = jnp.sum(hb * hb, axis=1)[:, None]
    cross = jax.lax.dot_general(hb, h, (((1,), (1,)), ((), ())),
                                precision=_DP)
    d2 = (sq_b + sq_all) - 2.0 * cross
    r0 = pl.program_id(1) * _RB
    rows = r0 + jax.lax.broadcasted_iota(jnp.int32, (_RB, _N), 0)
    cols = jax.lax.broadcasted_iota(jnp.int32, (_RB, _N), 1)
    diag = rows == cols
    d2 = jnp.where(diag, d2 + 1e9, d2)
    hb2 = _RB // 2
    colf = cols[:hb2].astype(jnp.float32)

    def _round(state):
        d2p, accp = state
        mn = jnp.min(d2p, axis=1, keepdims=True)
        eq = d2p == mn
        amin = jnp.min(jnp.where(eq, colf, float(_N)), axis=1, keepdims=True)
        hit = colf == amin
        accp = accp + jnp.where(hit, 1.0 / _K, 0.0)
        d2p = jnp.where(hit, jnp.inf, d2p)
        return d2p, accp

    za = jnp.zeros((hb2, _N), jnp.float32)
    sa = (d2[:hb2], za)
    sb = (d2[hb2:], za)
    for _ in range(_K):
        sa = _round(sa)
        sb = _round(sb)
    m_ref[0] = jnp.concatenate([sa[1], sb[1]], axis=0).astype(jnp.bfloat16)


def _conv_kernel(h0_ref, m_ref, w1_ref, b1_ref, w2_ref, b2_ref, w3_ref,
                 b3_ref, out_ref):
    h = h0_ref[0]
    m = m_ref[0]
    dn = (((1,), (0,)), ((), ()))
    for w_ref, b_ref in ((w1_ref, b1_ref), (w2_ref, b2_ref), (w3_ref, b3_ref)):
        hh = h.astype(jnp.bfloat16)
        r1 = h - hh.astype(jnp.float32)
        hm = r1.astype(jnp.bfloat16)
        hl = (r1 - hm.astype(jnp.float32)).astype(jnp.bfloat16)
        agg = (jax.lax.dot_general(m, hh, dn, precision=_DP,
                                   preferred_element_type=jnp.float32)
               + jax.lax.dot_general(m, hm, dn, precision=_DP,
                                     preferred_element_type=jnp.float32)
               + jax.lax.dot_general(m, hl, dn, precision=_DP,
                                     preferred_element_type=jnp.float32))
        hp = h + agg
        h = jnp.tanh(jax.lax.dot_general(
            hp, w_ref[...], (((1,), (0,)), ((), ())),
            precision=_DP) + b_ref[...])
    out_ref[0] = h


def _pool_kernel(h_ref, out_ref):
    h = h_ref[0]
    ch = h.shape[1]
    e = (jax.lax.broadcasted_iota(jnp.int32, (1, ch), 1) == (ch - 1))
    e = e.astype(jnp.float32)
    scores = jax.lax.dot_general(e, h, (((1,), (1,)), ((), ())),
                                 precision=_HP)
    cols = jax.lax.broadcasted_iota(jnp.int32, (1, _N), 1).astype(jnp.float32)
    rows32 = jax.lax.broadcasted_iota(jnp.int32, (_GO, 1), 0).astype(jnp.float32)
    ranks = jnp.zeros((1, _N), jnp.float32)
    for r in range(_GO):
        mx = jnp.max(scores, axis=1, keepdims=True)
        eq = scores == mx
        amin = jnp.min(jnp.where(eq, cols, float(_N)), axis=1, keepdims=True)
        hit = cols == amin
        ranks = ranks + jnp.where(hit, float(r + 1), 0.0)
        scores = jnp.where(hit, -jnp.inf, scores)
    sel = (ranks == rows32 + 1.0).astype(jnp.float32)
    out_ref[0] = jax.lax.dot_general(sel, h, (((1,), (0,)), ((), ())),
                                     precision=_HP)


def _fc1_kernel(x_ref, w_ref, b_ref, out_ref):
    k = pl.program_id(0)

    @pl.when(k == 0)
    def _init():
        out_ref[...] = jnp.zeros_like(out_ref)

    out_ref[...] += jax.lax.dot_general(
        x_ref[...], w_ref[...], (((1,), (0,)), ((), ())), precision=_DP)

    @pl.when(k == pl.num_programs(0) - 1)
    def _fin():
        v = out_ref[...] + b_ref[...]
        out_ref[...] = jnp.where(v >= 0, v, 0.01 * v)


def _fc2_kernel(x_ref, w_ref, b_ref, out_ref):
    out_ref[...] = jax.lax.dot_general(
        x_ref[...], w_ref[...], (((1,), (0,)), ((), ())),
        precision=_DP) + b_ref[...]


def kernel(x, W1, b1, W2, b2, W3, b3, Wc1, bc1, Wc2, bc2):
    bsz = x.shape[0]
    h0 = x.reshape(bsz, _N, -1)
    c = h0.shape[2]
    ch = W1.shape[1]

    m = pl.pallas_call(
        _knn_adj_kernel,
        grid=(bsz, _N // _RB),
        in_specs=[pl.BlockSpec((1, _N, c), lambda b, r: (b, 0, 0)),
                  pl.BlockSpec((1, _RB, c), lambda b, r: (b, r, 0))],
        out_specs=pl.BlockSpec((1, _RB, _N), lambda b, r: (b, r, 0)),
        out_shape=jax.ShapeDtypeStruct((bsz, _N, _N), jnp.bfloat16),
    )(h0, h0)

    h3 = pl.pallas_call(
        _conv_kernel,
        grid=(bsz,),
        in_specs=[pl.BlockSpec((1, _N, c), lambda b: (b, 0, 0)),
                  pl.BlockSpec((1, _N, _N), lambda b: (b, 0, 0)),
                  pl.BlockSpec(W1.shape, lambda b: (0, 0)),
                  pl.BlockSpec((1, ch), lambda b: (0, 0)),
                  pl.BlockSpec(W2.shape, lambda b: (0, 0)),
                  pl.BlockSpec((1, ch), lambda b: (0, 0)),
                  pl.BlockSpec(W3.shape, lambda b: (0, 0)),
                  pl.BlockSpec((1, ch), lambda b: (0, 0))],
        out_specs=pl.BlockSpec((1, _N, ch), lambda b: (b, 0, 0)),
        out_shape=jax.ShapeDtypeStruct((bsz, _N, ch), jnp.float32),
    )(h0, m, W1, b1.reshape(1, ch), W2, b2.reshape(1, ch),
      W3, b3.reshape(1, ch))

    pooled = pl.pallas_call(
        _pool_kernel,
        grid=(bsz,),
        in_specs=[pl.BlockSpec((1, _N, ch), lambda b: (b, 0, 0))],
        out_specs=pl.BlockSpec((1, _GO, ch), lambda b: (b, 0, 0)),
        out_shape=jax.ShapeDtypeStruct((bsz, _GO, ch), jnp.float32),
    )(h3)

    flat = pooled.reshape(bsz, _GO * ch)
    fc_in, fc_mid = Wc1.shape
    kn = 8
    kblk = fc_in // kn

    fc1 = pl.pallas_call(
        _fc1_kernel,
        grid=(kn,),
        in_specs=[pl.BlockSpec((bsz, kblk), lambda k: (0, k)),
                  pl.BlockSpec((kblk, fc_mid), lambda k: (k, 0)),
                  pl.BlockSpec((1, fc_mid), lambda k: (0, 0))],
        out_specs=pl.BlockSpec((bsz, fc_mid), lambda k: (0, 0)),
        out_shape=jax.ShapeDtypeStruct((bsz, fc_mid), jnp.float32),
    )(flat, Wc1, bc1.reshape(1, fc_mid))

    nclass = Wc2.shape[1]
    out = pl.pallas_call(
        _fc2_kernel,
        in_specs=[pl.BlockSpec((bsz, fc_mid), lambda: (0, 0)),
                  pl.BlockSpec(Wc2.shape, lambda: (0, 0)),
                  pl.BlockSpec((1, nclass), lambda: (0, 0))],
        out_specs=pl.BlockSpec((bsz, nclass), lambda: (0, 0)),
        out_shape=jax.ShapeDtypeStruct((bsz, nclass), jnp.float32),
    )(fc1, Wc2, bc2.reshape(1, nclass))
    return out


# speedup vs baseline: 1.0841x; 1.0780x over previous
import jax
import jax.numpy as jnp
from jax.experimental import pallas as pl

_N = 1024
_K = 16
_RB = 256
_GO = 32
_HP = jax.lax.Precision.HIGHEST
_DP = jax.lax.Precision.DEFAULT


def _knn_adj_kernel(hall_ref, hrow_ref, m_ref):
    h = hall_ref[0]
    hb = hrow_ref[0]
    sq_all = jnp.sum(h * h, axis=1)[None, :]
    sq_b = jnp.sum(hb * hb, axis=1)[:, None]
    cross = jax.lax.dot_general(hb, h, (((1,), (1,)), ((), ())),
                                precision=_DP)
    d2 = (sq_b + sq_all) - 2.0 * cross
    r0 = pl.program_id(1) * _RB
    rows = r0 + jax.lax.broadcasted_iota(jnp.int32, (_RB, _N), 0)
    cols = jax.lax.broadcasted_iota(jnp.int32, (_RB, _N), 1)
    diag = rows == cols
    d2 = jnp.where(diag, d2 + 1e9, d2)
    hb2 = _RB // 2
    colf = cols[:hb2].astype(jnp.float32)

    def _round(state):
        d2p, accp = state
        mn = jnp.min(d2p, axis=1, keepdims=True)
        eq = d2p == mn
        amin = jnp.min(jnp.where(eq, colf, float(_N)), axis=1, keepdims=True)
        hit = colf == amin
        accp = accp + jnp.where(hit, 1.0 / _K, 0.0)
        d2p = jnp.where(hit, jnp.inf, d2p)
        return d2p, accp

    za = jnp.zeros((hb2, _N), jnp.float32)
    sa = (d2[:hb2], za)
    sb = (d2[hb2:], za)
    for _ in range(_K):
        sa = _round(sa)
        sb = _round(sb)
    m_ref[0] = jnp.concatenate([sa[1], sb[1]], axis=0).astype(jnp.bfloat16)


def _conv_kernel(h0_ref, m_ref, w1_ref, b1_ref, w2_ref, b2_ref, w3_ref,
                 b3_ref, out_ref):
    h = h0_ref[0]
    m = m_ref[0]
    dn = (((1,), (0,)), ((), ()))
    for w_ref, b_ref in ((w1_ref, b1_ref), (w2_ref, b2_ref), (w3_ref, b3_ref)):
        hh = h.astype(jnp.bfloat16)
        r1 = h - hh.astype(jnp.float32)
        hm = r1.astype(jnp.bfloat16)
        hl = (r1 - hm.astype(jnp.float32)).astype(jnp.bfloat16)
        agg = (jax.lax.dot_general(m, hh, dn, precision=_DP,
                                   preferred_element_type=jnp.float32)
               + jax.lax.dot_general(m, hm, dn, precision=_DP,
                                     preferred_element_type=jnp.float32)
               + jax.lax.dot_general(m, hl, dn, precision=_DP,
                                     preferred_element_type=jnp.float32))
        hp = h + agg
        h = jnp.tanh(jax.lax.dot_general(
            hp, w_ref[...], (((1,), (0,)), ((), ())),
            precision=_DP) + b_ref[...])
    out_ref[0] = h


def _pool_kernel(h_ref, out_ref):
    h = h_ref[...]
    bsz, _, ch = h.shape
    e = (jax.lax.broadcasted_iota(jnp.int32, (1, ch), 1) == (ch - 1))
    e = e.astype(jnp.float32)
    scores = jnp.concatenate(
        [jax.lax.dot_general(e, h[b], (((1,), (1,)), ((), ())), precision=_HP)
         for b in range(bsz)], axis=0)
    cols = jax.lax.broadcasted_iota(jnp.int32, (bsz, _N), 1).astype(jnp.float32)
    rows32 = jax.lax.broadcasted_iota(jnp.int32, (_GO, 1), 0).astype(jnp.float32)
    ranks = jnp.zeros((bsz, _N), jnp.float32)
    for r in range(_GO):
        mx = jnp.max(scores, axis=1, keepdims=True)
        eq = scores == mx
        amin = jnp.min(jnp.where(eq, cols, float(_N)), axis=1, keepdims=True)
        hit = cols == amin
        ranks = ranks + jnp.where(hit, float(r + 1), 0.0)
        scores = jnp.where(hit, -jnp.inf, scores)
    for b in range(bsz):
        sel = (ranks[b:b + 1] == rows32 + 1.0).astype(jnp.float32)
        out_ref[b] = jax.lax.dot_general(sel, h[b], (((1,), (0,)), ((), ())),
                                         precision=_HP)


def _fc1_kernel(x_ref, w_ref, b_ref, out_ref):
    k = pl.program_id(0)

    @pl.when(k == 0)
    def _init():
        out_ref[...] = jnp.zeros_like(out_ref)

    out_ref[...] += jax.lax.dot_general(
        x_ref[...], w_ref[...], (((1,), (0,)), ((), ())), precision=_DP)

    @pl.when(k == pl.num_programs(0) - 1)
    def _fin():
        v = out_ref[...] + b_ref[...]
        out_ref[...] = jnp.where(v >= 0, v, 0.01 * v)


def _fc2_kernel(x_ref, w_ref, b_ref, out_ref):
    out_ref[...] = jax.lax.dot_general(
        x_ref[...], w_ref[...], (((1,), (0,)), ((), ())),
        precision=_DP) + b_ref[...]


def kernel(x, W1, b1, W2, b2, W3, b3, Wc1, bc1, Wc2, bc2):
    bsz = x.shape[0]
    h0 = x.reshape(bsz, _N, -1)
    c = h0.shape[2]
    ch = W1.shape[1]

    m = pl.pallas_call(
        _knn_adj_kernel,
        grid=(bsz, _N // _RB),
        in_specs=[pl.BlockSpec((1, _N, c), lambda b, r: (b, 0, 0)),
                  pl.BlockSpec((1, _RB, c), lambda b, r: (b, r, 0))],
        out_specs=pl.BlockSpec((1, _RB, _N), lambda b, r: (b, r, 0)),
        out_shape=jax.ShapeDtypeStruct((bsz, _N, _N), jnp.bfloat16),
    )(h0, h0)

    h3 = pl.pallas_call(
        _conv_kernel,
        grid=(bsz,),
        in_specs=[pl.BlockSpec((1, _N, c), lambda b: (b, 0, 0)),
                  pl.BlockSpec((1, _N, _N), lambda b: (b, 0, 0)),
                  pl.BlockSpec(W1.shape, lambda b: (0, 0)),
                  pl.BlockSpec((1, ch), lambda b: (0, 0)),
                  pl.BlockSpec(W2.shape, lambda b: (0, 0)),
                  pl.BlockSpec((1, ch), lambda b: (0, 0)),
                  pl.BlockSpec(W3.shape, lambda b: (0, 0)),
                  pl.BlockSpec((1, ch), lambda b: (0, 0))],
        out_specs=pl.BlockSpec((1, _N, ch), lambda b: (b, 0, 0)),
        out_shape=jax.ShapeDtypeStruct((bsz, _N, ch), jnp.float32),
    )(h0, m, W1, b1.reshape(1, ch), W2, b2.reshape(1, ch),
      W3, b3.reshape(1, ch))

    pooled = pl.pallas_call(
        _pool_kernel,
        in_specs=[pl.BlockSpec((bsz, _N, ch), lambda: (0, 0, 0))],
        out_specs=pl.BlockSpec((bsz, _GO, ch), lambda: (0, 0, 0)),
        out_shape=jax.ShapeDtypeStruct((bsz, _GO, ch), jnp.float32),
    )(h3)

    flat = pooled.reshape(bsz, _GO * ch)
    fc_in, fc_mid = Wc1.shape
    kn = 8
    kblk = fc_in // kn

    fc1 = pl.pallas_call(
        _fc1_kernel,
        grid=(kn,),
        in_specs=[pl.BlockSpec((bsz, kblk), lambda k: (0, k)),
                  pl.BlockSpec((kblk, fc_mid), lambda k: (k, 0)),
                  pl.BlockSpec((1, fc_mid), lambda k: (0, 0))],
        out_specs=pl.BlockSpec((bsz, fc_mid), lambda k: (0, 0)),
        out_shape=jax.ShapeDtypeStruct((bsz, fc_mid), jnp.float32),
    )(flat, Wc1, bc1.reshape(1, fc_mid))

    nclass = Wc2.shape[1]
    out = pl.pallas_call(
        _fc2_kernel,
        in_specs=[pl.BlockSpec((bsz, fc_mid), lambda: (0, 0)),
                  pl.BlockSpec(Wc2.shape, lambda: (0, 0)),
                  pl.BlockSpec((1, nclass), lambda: (0, 0))],
        out_specs=pl.BlockSpec((bsz, nclass), lambda: (0, 0)),
        out_shape=jax.ShapeDtypeStruct((bsz, nclass), jnp.float32),
    )(fc1, Wc2, bc2.reshape(1, nclass))
    return out


# speedup vs baseline: 1.0906x; 1.0060x over previous
import jax
import jax.numpy as jnp
from jax.experimental import pallas as pl

_N = 1024
_K = 16
_RB = 256
_GO = 32
_HP = jax.lax.Precision.HIGHEST
_DP = jax.lax.Precision.DEFAULT


def _knn_adj_kernel(hall_ref, hrow_ref, m_ref):
    h = hall_ref[0]
    hb = hrow_ref[0]
    sq_all = jnp.sum(h * h, axis=1)[None, :]
    sq_b = jnp.sum(hb * hb, axis=1)[:, None]
    cross = jax.lax.dot_general(hb, h, (((1,), (1,)), ((), ())),
                                precision=_DP)
    d2 = (sq_b + sq_all) - 2.0 * cross
    r0 = pl.program_id(1) * _RB
    rows = r0 + jax.lax.broadcasted_iota(jnp.int32, (_RB, _N), 0)
    cols = jax.lax.broadcasted_iota(jnp.int32, (_RB, _N), 1)
    diag = rows == cols
    d2 = jnp.where(diag, d2 + 1e9, d2)
    hb2 = _RB // 2
    colf = cols[:hb2].astype(jnp.float32)

    def _round(state):
        d2p, accp = state
        mn = jnp.min(d2p, axis=1, keepdims=True)
        eq = d2p == mn
        amin = jnp.min(jnp.where(eq, colf, float(_N)), axis=1, keepdims=True)
        hit = colf == amin
        accp = accp + jnp.where(hit, 1.0 / _K, 0.0)
        d2p = jnp.where(hit, jnp.inf, d2p)
        return d2p, accp

    za = jnp.zeros((hb2, _N), jnp.float32)
    sa = (d2[:hb2], za)
    sb = (d2[hb2:], za)
    for _ in range(_K):
        sa = _round(sa)
        sb = _round(sb)
    m_ref[0] = jnp.concatenate([sa[1], sb[1]], axis=0).astype(jnp.bfloat16)


def _conv_kernel(h0_ref, m_ref, w1_ref, b1_ref, w2_ref, b2_ref, w3_ref,
                 b3_ref, out_ref):
    h = h0_ref[0]
    m = m_ref[0]
    dn = (((1,), (0,)), ((), ()))
    for w_ref, b_ref in ((w1_ref, b1_ref), (w2_ref, b2_ref), (w3_ref, b3_ref)):
        hh = h.astype(jnp.bfloat16)
        r1 = h - hh.astype(jnp.float32)
        hm = r1.astype(jnp.bfloat16)
        hl = (r1 - hm.astype(jnp.float32)).astype(jnp.bfloat16)
        agg = (jax.lax.dot_general(m, hh, dn, precision=_DP,
                                   preferred_element_type=jnp.float32)
               + jax.lax.dot_general(m, hm, dn, precision=_DP,
                                     preferred_element_type=jnp.float32)
               + jax.lax.dot_general(m, hl, dn, precision=_DP,
                                     preferred_element_type=jnp.float32))
        hp = h + agg
        h = jnp.tanh(jax.lax.dot_general(
            hp, w_ref[...], (((1,), (0,)), ((), ())),
            precision=_DP) + b_ref[...])
    out_ref[0] = h


def _pool_kernel(h_ref, out_ref):
    h = h_ref[...]
    bsz, _, ch = h.shape
    e = (jax.lax.broadcasted_iota(jnp.int32, (1, ch), 1) == (ch - 1))
    e = e.astype(jnp.float32)
    scores = jnp.concatenate(
        [jax.lax.dot_general(e, h[b], (((1,), (1,)), ((), ())), precision=_HP)
         for b in range(bsz)], axis=0)
    cols = jax.lax.broadcasted_iota(jnp.int32, (bsz, _N), 1).astype(jnp.float32)
    rows32 = jax.lax.broadcasted_iota(jnp.int32, (_GO, 1), 0).astype(jnp.float32)
    ranks = jnp.zeros((bsz, _N), jnp.float32)
    for r in range(_GO):
        mx = jnp.max(scores, axis=1, keepdims=True)
        eq = scores == mx
        amin = jnp.min(jnp.where(eq, cols, float(_N)), axis=1, keepdims=True)
        hit = cols == amin
        ranks = ranks + jnp.where(hit, float(r + 1), 0.0)
        scores = jnp.where(hit, -jnp.inf, scores)
    for b in range(bsz):
        sel = (ranks[b:b + 1] == rows32 + 1.0).astype(jnp.float32)
        out_ref[b] = jax.lax.dot_general(sel, h[b], (((1,), (0,)), ((), ())),
                                         precision=_HP)


def _fc1_kernel(x_ref, w_ref, b_ref, out_ref):
    k = pl.program_id(0)

    @pl.when(k == 0)
    def _init():
        out_ref[...] = jnp.zeros_like(out_ref)

    out_ref[...] += jax.lax.dot_general(
        x_ref[...], w_ref[...], (((1,), (0,)), ((), ())), precision=_DP)

    @pl.when(k == pl.num_programs(0) - 1)
    def _fin():
        v = out_ref[...] + b_ref[...]
        out_ref[...] = jnp.where(v >= 0, v, 0.01 * v)


def _fc2_kernel(x_ref, w_ref, b_ref, out_ref):
    out_ref[...] = jax.lax.dot_general(
        x_ref[...], w_ref[...], (((1,), (0,)), ((), ())),
        precision=_DP) + b_ref[...]


def kernel(x, W1, b1, W2, b2, W3, b3, Wc1, bc1, Wc2, bc2):
    bsz = x.shape[0]
    h0 = x.reshape(bsz, _N, -1)
    c = h0.shape[2]
    ch = W1.shape[1]

    m = pl.pallas_call(
        _knn_adj_kernel,
        grid=(bsz, _N // _RB),
        in_specs=[pl.BlockSpec((1, _N, c), lambda b, r: (b, 0, 0)),
                  pl.BlockSpec((1, _RB, c), lambda b, r: (b, r, 0))],
        out_specs=pl.BlockSpec((1, _RB, _N), lambda b, r: (b, r, 0)),
        out_shape=jax.ShapeDtypeStruct((bsz, _N, _N), jnp.bfloat16),
    )(h0, h0)

    h3 = pl.pallas_call(
        _conv_kernel,
        grid=(bsz,),
        in_specs=[pl.BlockSpec((1, _N, c), lambda b: (b, 0, 0)),
                  pl.BlockSpec((1, _N, _N), lambda b: (b, 0, 0)),
                  pl.BlockSpec(W1.shape, lambda b: (0, 0)),
                  pl.BlockSpec((1, ch), lambda b: (0, 0)),
                  pl.BlockSpec(W2.shape, lambda b: (0, 0)),
                  pl.BlockSpec((1, ch), lambda b: (0, 0)),
                  pl.BlockSpec(W3.shape, lambda b: (0, 0)),
                  pl.BlockSpec((1, ch), lambda b: (0, 0))],
        out_specs=pl.BlockSpec((1, _N, ch), lambda b: (b, 0, 0)),
        out_shape=jax.ShapeDtypeStruct((bsz, _N, ch), jnp.float32),
    )(h0, m, W1, b1.reshape(1, ch), W2, b2.reshape(1, ch),
      W3, b3.reshape(1, ch))

    pooled = pl.pallas_call(
        _pool_kernel,
        in_specs=[pl.BlockSpec((bsz, _N, ch), lambda: (0, 0, 0))],
        out_specs=pl.BlockSpec((bsz, _GO, ch), lambda: (0, 0, 0)),
        out_shape=jax.ShapeDtypeStruct((bsz, _GO, ch), jnp.float32),
    )(h3)

    flat = pooled.reshape(bsz, _GO * ch)
    fc_in, fc_mid = Wc1.shape
    kn = 16
    kblk = fc_in // kn

    fc1 = pl.pallas_call(
        _fc1_kernel,
        grid=(kn,),
        in_specs=[pl.BlockSpec((bsz, kblk), lambda k: (0, k)),
                  pl.BlockSpec((kblk, fc_mid), lambda k: (k, 0)),
                  pl.BlockSpec((1, fc_mid), lambda k: (0, 0))],
        out_specs=pl.BlockSpec((bsz, fc_mid), lambda k: (0, 0)),
        out_shape=jax.ShapeDtypeStruct((bsz, fc_mid), jnp.float32),
    )(flat, Wc1, bc1.reshape(1, fc_mid))

    nclass = Wc2.shape[1]
    out = pl.pallas_call(
        _fc2_kernel,
        in_specs=[pl.BlockSpec((bsz, fc_mid), lambda: (0, 0)),
                  pl.BlockSpec(Wc2.shape, lambda: (0, 0)),
                  pl.BlockSpec((1, nclass), lambda: (0, 0))],
        out_specs=pl.BlockSpec((bsz, nclass), lambda: (0, 0)),
        out_shape=jax.ShapeDtypeStruct((bsz, nclass), jnp.float32),
    )(fc1, Wc2, bc2.reshape(1, nclass))
    return out
